# baseline (speedup 1.0000x reference)
import functools

import jax
import jax.numpy as jnp
from jax import lax
from jax.experimental import pallas as pl
from jax.experimental.pallas import tpu as pltpu
from jax.experimental.pallas import tpu_sc as plsc

_B, _F, _V, _D = 4096, 26, 100000, 64
_N = _B * _F
_NC, _NS = 2, 16
_NW = _NC * _NS
_DPW = _D // _NW
_GRP = _B // 16


def _make_gather():
    mesh = plsc.VectorSubcoreMesh(core_axis_name="c", subcore_axis_name="s")

    @functools.partial(
        pl.kernel,
        mesh=mesh,
        compiler_params=pltpu.CompilerParams(
            use_tc_tiling_on_sc=False, needs_layout_passes=False),
        out_type=jax.ShapeDtypeStruct((_F, _D, _B), jnp.float32),
        scratch_types=[
            pltpu.VMEM((_V,), jnp.float32),
            pltpu.VMEM((2, _B), jnp.int32),
            pltpu.VMEM((2, _B), jnp.float32),
            pltpu.SemaphoreType.DMA,
            pltpu.SemaphoreType.DMA,
            pltpu.SemaphoreType.DMA,
            pltpu.SemaphoreType.DMA,
        ],
    )
    def gather(idx_hbm, table_hbm, out_hbm, row_v, idx_v, val_v,
               i0, i1, o0, o1):
        wid = lax.axis_index("s") * _NC + lax.axis_index("c")
        isem = (i0, i1)
        osem = (o0, o1)
        for r in range(_DPW):
            d = wid * _DPW + r
            pltpu.sync_copy(table_hbm.at[d], row_v)
            ih = [None, None]
            oh = [None, None]
            ih[0] = pltpu.async_copy(
                idx_hbm.at[pl.ds(0, _B)], idx_v.at[0], isem[0])
            for f in range(_F):
                bi = f % 2
                ni = (f + 1) % 2
                if f + 1 < _F:
                    ih[ni] = pltpu.async_copy(
                        idx_hbm.at[pl.ds((f + 1) * _B, _B)],
                        idx_v.at[ni], isem[ni])
                ih[bi].wait()
                if oh[bi] is not None:
                    oh[bi].wait()

                def body(g, _):
                    iv = idx_v[bi, pl.ds(g * 16, 16)]
                    val_v[bi, pl.ds(g * 16, 16)] = plsc.load_gather(
                        row_v, [iv])
                    return _

                lax.fori_loop(0, _GRP, body, 0, unroll=8)
                oh[bi] = pltpu.async_copy(
                    val_v.at[bi], out_hbm.at[f].at[d], osem[bi])
            for bi in range(2):
                if oh[bi] is not None:
                    oh[bi].wait()

    return gather


_make_gather = functools.cache(_make_gather)

_BB = 512


def _dense_body(x_ref, wd_ref, bd_ref, wcm_ref, wcf_ref, bc_ref,
                wsm_ref, wsf_ref, bs_ref, outc_ref, outs_ref):
    deep = jnp.zeros((16, _BB), dtype=jnp.float32)
    t1 = jnp.zeros((_D, _BB), dtype=jnp.float32)
    t2 = jnp.zeros((_D, _BB), dtype=jnp.float32)
    for f in range(_F):
        xf = x_ref[f]
        deep = deep + jnp.dot(wd_ref[f], xf,
                              preferred_element_type=jnp.float32)
        t1 = t1 + xf
        t2 = t2 + xf * xf
    deep = jnp.maximum(deep + bd_ref[...], 0.0)
    s1 = jnp.sum(t1, axis=0, keepdims=True)
    s2 = jnp.sum(t2, axis=0, keepdims=True)
    cross = 0.5 * (s1 * s1 - s2)
    zc = (jnp.dot(wcm_ref[...], deep, preferred_element_type=jnp.float32)
          + wcf_ref[...] * cross + bc_ref[...])
    outc_ref[...] = 1.0 / (1.0 + jnp.exp(-zc))
    zs = (jnp.dot(wsm_ref[...], deep, preferred_element_type=jnp.float32)
          + wsf_ref[...] * cross + bs_ref[...])
    outs_ref[...] = 1.0 / (1.0 + jnp.exp(-zs))


def _dense(x, wdT, bdT, wcmT, wcfT, bcT, wsmT, wsfT, bsT):
    grid = (_B // _BB,)
    full2 = lambda shape: pl.BlockSpec(shape, lambda i: (0, 0))
    return pl.pallas_call(
        _dense_body,
        grid=grid,
        in_specs=[
            pl.BlockSpec((_F, _D, _BB), lambda i: (0, 0, i)),
            pl.BlockSpec(wdT.shape, lambda i: (0, 0, 0)),
            full2(bdT.shape),
            full2(wcmT.shape),
            full2(wcfT.shape),
            full2(bcT.shape),
            full2(wsmT.shape),
            full2(wsfT.shape),
            full2(bsT.shape),
        ],
        out_specs=[
            pl.BlockSpec((728, _BB), lambda i: (0, i)),
            pl.BlockSpec((92, _BB), lambda i: (0, i)),
        ],
        out_shape=[
            jax.ShapeDtypeStruct((728, _B), jnp.float32),
            jax.ShapeDtypeStruct((92, _B), jnp.float32),
        ],
    )(x, wdT, bdT, wcmT, wcfT, bcT, wsmT, wsfT, bsT)


def kernel(indices, emb_table, w_deep, b_deep, w_course, b_course, w_sub, b_sub):
    idx_fm = indices.astype(jnp.int32).T.reshape(-1)
    t_cm = emb_table.T
    x = _make_gather()(idx_fm, t_cm)
    wdT = jnp.transpose(w_deep.reshape(_F, _D, 16), (0, 2, 1))
    logits_cT, logits_sT = _dense(
        x,
        wdT,
        b_deep.reshape(16, 1),
        w_course[:16].T,
        w_course[16].reshape(728, 1),
        b_course.reshape(728, 1),
        w_sub[:16].T,
        w_sub[16].reshape(92, 1),
        b_sub.reshape(92, 1),
    )
    return (logits_cT.T, logits_sT.T)


# speedup vs baseline: 1.6911x; 1.6911x over previous
import functools

import jax
import jax.numpy as jnp
from jax import lax
from jax.experimental import pallas as pl
from jax.experimental.pallas import tpu as pltpu
from jax.experimental.pallas import tpu_sc as plsc

_B, _F, _V, _D = 4096, 26, 100000, 64
_N = _B * _F
_NC, _NS = 2, 16
_NW = _NC * _NS
_DPW = _D // _NW
_GRP = _B // 16
_UNROLL = 16


def _make_gather():
    mesh = plsc.VectorSubcoreMesh(core_axis_name="c", subcore_axis_name="s")

    @functools.partial(
        pl.kernel,
        mesh=mesh,
        compiler_params=pltpu.CompilerParams(needs_layout_passes=False),
        out_type=jax.ShapeDtypeStruct((_F, _D, _B), jnp.float32),
        scratch_types=[
            pltpu.VMEM((_V,), jnp.float32),
            pltpu.VMEM((2, _B), jnp.int32),
            pltpu.VMEM((2, _B), jnp.float32),
            pltpu.SemaphoreType.DMA,
            pltpu.SemaphoreType.DMA,
            pltpu.SemaphoreType.DMA,
            pltpu.SemaphoreType.DMA,
        ],
    )
    def gather(idx_hbm, table_hbm, out_hbm, row_v, idx_v, val_v,
               i0, i1, o0, o1):
        wid = lax.axis_index("s") * _NC + lax.axis_index("c")
        isem = (i0, i1)
        osem = (o0, o1)
        for r in range(_DPW):
            d = wid * _DPW + r
            pltpu.sync_copy(table_hbm.at[d], row_v)
            ih = [None, None]
            oh = [None, None]
            ih[0] = pltpu.async_copy(
                idx_hbm.at[pl.ds(0, _B)], idx_v.at[0], isem[0])
            for f in range(_F):
                bi = f % 2
                ni = (f + 1) % 2
                if f + 1 < _F:
                    ih[ni] = pltpu.async_copy(
                        idx_hbm.at[pl.ds((f + 1) * _B, _B)],
                        idx_v.at[ni], isem[ni])
                ih[bi].wait()
                if oh[bi] is not None:
                    oh[bi].wait()

                def body(h, _):
                    base = h * (16 * _UNROLL)
                    for k in range(_UNROLL):
                        off = base + k * 16
                        iv = idx_v[bi, pl.ds(off, 16)]
                        val_v[bi, pl.ds(off, 16)] = plsc.load_gather(
                            row_v, [iv])
                    return _

                lax.fori_loop(0, _GRP // _UNROLL, body, 0)
                oh[bi] = pltpu.async_copy(
                    val_v.at[bi], out_hbm.at[f].at[d], osem[bi])
            for bi in range(2):
                if oh[bi] is not None:
                    oh[bi].wait()

    return gather


_make_gather = functools.cache(_make_gather)

_BB = 512


def _dense_body(x_ref, wd_ref, bd_ref, wcm_ref, wcf_ref, bc_ref,
                wsm_ref, wsf_ref, bs_ref, outc_ref, outs_ref):
    deep = jnp.zeros((16, _BB), dtype=jnp.float32)
    t1 = jnp.zeros((_D, _BB), dtype=jnp.float32)
    t2 = jnp.zeros((_D, _BB), dtype=jnp.float32)
    for f in range(_F):
        xf = x_ref[f]
        deep = deep + jnp.dot(wd_ref[f], xf,
                              preferred_element_type=jnp.float32)
        t1 = t1 + xf
        t2 = t2 + xf * xf
    deep = jnp.maximum(deep + bd_ref[...], 0.0)
    s1 = jnp.sum(t1, axis=0, keepdims=True)
    s2 = jnp.sum(t2, axis=0, keepdims=True)
    cross = 0.5 * (s1 * s1 - s2)
    zc = (jnp.dot(wcm_ref[...], deep, preferred_element_type=jnp.float32)
          + wcf_ref[...] * cross + bc_ref[...])
    outc_ref[...] = 1.0 / (1.0 + jnp.exp(-zc))
    zs = (jnp.dot(wsm_ref[...], deep, preferred_element_type=jnp.float32)
          + wsf_ref[...] * cross + bs_ref[...])
    outs_ref[...] = 1.0 / (1.0 + jnp.exp(-zs))


def _dense(x, wdT, bdT, wcmT, wcfT, bcT, wsmT, wsfT, bsT):
    grid = (_B // _BB,)
    full2 = lambda shape: pl.BlockSpec(shape, lambda i: (0, 0))
    return pl.pallas_call(
        _dense_body,
        grid=grid,
        in_specs=[
            pl.BlockSpec((_F, _D, _BB), lambda i: (0, 0, i)),
            pl.BlockSpec(wdT.shape, lambda i: (0, 0, 0)),
            full2(bdT.shape),
            full2(wcmT.shape),
            full2(wcfT.shape),
            full2(bcT.shape),
            full2(wsmT.shape),
            full2(wsfT.shape),
            full2(bsT.shape),
        ],
        out_specs=[
            pl.BlockSpec((728, _BB), lambda i: (0, i)),
            pl.BlockSpec((92, _BB), lambda i: (0, i)),
        ],
        out_shape=[
            jax.ShapeDtypeStruct((728, _B), jnp.float32),
            jax.ShapeDtypeStruct((92, _B), jnp.float32),
        ],
    )(x, wdT, bdT, wcmT, wcfT, bcT, wsmT, wsfT, bsT)


def kernel(indices, emb_table, w_deep, b_deep, w_course, b_course, w_sub, b_sub):
    idx_fm = indices.astype(jnp.int32).T.reshape(-1)
    t_cm = emb_table.T
    x = _make_gather()(idx_fm, t_cm)
    wdT = jnp.transpose(w_deep.reshape(_F, _D, 16), (0, 2, 1))
    logits_cT, logits_sT = _dense(
        x,
        wdT,
        b_deep.reshape(16, 1),
        w_course[:16].T,
        w_course[16].reshape(728, 1),
        b_course.reshape(728, 1),
        w_sub[:16].T,
        w_sub[16].reshape(92, 1),
        b_sub.reshape(92, 1),
    )
    return (logits_cT.T, logits_sT.T)


# speedup vs baseline: 2.1051x; 1.2448x over previous
import functools

import jax
import jax.numpy as jnp
from jax import lax
from jax.experimental import pallas as pl
from jax.experimental.pallas import tpu as pltpu
from jax.experimental.pallas import tpu_sc as plsc

_B, _F, _V, _D = 4096, 26, 100000, 64
_N = _B * _F
_NC, _NS = 2, 16
_NW = _NC * _NS
_DPW = _D // _NW
_GRP = _B // 16
_UNROLL = 16


def _make_gather():
    mesh = plsc.VectorSubcoreMesh(core_axis_name="c", subcore_axis_name="s")

    @functools.partial(
        pl.kernel,
        mesh=mesh,
        compiler_params=pltpu.CompilerParams(needs_layout_passes=False),
        out_type=jax.ShapeDtypeStruct((_F, _D, _B), jnp.float32),
        scratch_types=[
            pltpu.VMEM((_V,), jnp.float32),
            pltpu.VMEM((2, _B), jnp.int32),
            pltpu.VMEM((2, _B), jnp.float32),
            pltpu.SemaphoreType.DMA,
            pltpu.SemaphoreType.DMA,
            pltpu.SemaphoreType.DMA,
            pltpu.SemaphoreType.DMA,
        ],
    )
    def gather(idx_hbm, table_hbm, out_hbm, row_v, idx_v, val_v,
               i0, i1, o0, o1):
        wid = lax.axis_index("s") * _NC + lax.axis_index("c")
        isem = (i0, i1)
        osem = (o0, o1)
        for r in range(_DPW):
            d = wid * _DPW + r
            pltpu.sync_copy(table_hbm.at[d], row_v)
            ih = [None, None]
            oh = [None, None]
            ih[0] = pltpu.async_copy(
                idx_hbm.at[pl.ds(0, _B)], idx_v.at[0], isem[0])
            for f in range(_F):
                bi = f % 2
                ni = (f + 1) % 2
                if f + 1 < _F:
                    ih[ni] = pltpu.async_copy(
                        idx_hbm.at[pl.ds((f + 1) * _B, _B)],
                        idx_v.at[ni], isem[ni])
                ih[bi].wait()
                if oh[bi] is not None:
                    oh[bi].wait()

                @plsc.parallel_loop(0, _GRP, 1, unroll=_UNROLL)
                def _gather_body(g):
                    off = g * 16
                    iv = idx_v[bi, pl.ds(off, 16)]
                    val_v[bi, pl.ds(off, 16)] = plsc.load_gather(
                        row_v, [iv])
                oh[bi] = pltpu.async_copy(
                    val_v.at[bi], out_hbm.at[f].at[d], osem[bi])
            for bi in range(2):
                if oh[bi] is not None:
                    oh[bi].wait()

    return gather


_make_gather = functools.cache(_make_gather)

_BB = 512


def _dense_body(x_ref, wd_ref, bd_ref, wcm_ref, wcf_ref, bc_ref,
                wsm_ref, wsf_ref, bs_ref, outc_ref, outs_ref):
    deep = jnp.zeros((16, _BB), dtype=jnp.float32)
    t1 = jnp.zeros((_D, _BB), dtype=jnp.float32)
    t2 = jnp.zeros((_D, _BB), dtype=jnp.float32)
    for f in range(_F):
        xf = x_ref[f]
        deep = deep + jnp.dot(wd_ref[f], xf,
                              preferred_element_type=jnp.float32)
        t1 = t1 + xf
        t2 = t2 + xf * xf
    deep = jnp.maximum(deep + bd_ref[...], 0.0)
    s1 = jnp.sum(t1, axis=0, keepdims=True)
    s2 = jnp.sum(t2, axis=0, keepdims=True)
    cross = 0.5 * (s1 * s1 - s2)
    zc = (jnp.dot(wcm_ref[...], deep, preferred_element_type=jnp.float32)
          + wcf_ref[...] * cross + bc_ref[...])
    outc_ref[...] = 1.0 / (1.0 + jnp.exp(-zc))
    zs = (jnp.dot(wsm_ref[...], deep, preferred_element_type=jnp.float32)
          + wsf_ref[...] * cross + bs_ref[...])
    outs_ref[...] = 1.0 / (1.0 + jnp.exp(-zs))


def _dense(x, wdT, bdT, wcmT, wcfT, bcT, wsmT, wsfT, bsT):
    grid = (_B // _BB,)
    full2 = lambda shape: pl.BlockSpec(shape, lambda i: (0, 0))
    return pl.pallas_call(
        _dense_body,
        grid=grid,
        in_specs=[
            pl.BlockSpec((_F, _D, _BB), lambda i: (0, 0, i)),
            pl.BlockSpec(wdT.shape, lambda i: (0, 0, 0)),
            full2(bdT.shape),
            full2(wcmT.shape),
            full2(wcfT.shape),
            full2(bcT.shape),
            full2(wsmT.shape),
            full2(wsfT.shape),
            full2(bsT.shape),
        ],
        out_specs=[
            pl.BlockSpec((728, _BB), lambda i: (0, i)),
            pl.BlockSpec((92, _BB), lambda i: (0, i)),
        ],
        out_shape=[
            jax.ShapeDtypeStruct((728, _B), jnp.float32),
            jax.ShapeDtypeStruct((92, _B), jnp.float32),
        ],
    )(x, wdT, bdT, wcmT, wcfT, bcT, wsmT, wsfT, bsT)


def kernel(indices, emb_table, w_deep, b_deep, w_course, b_course, w_sub, b_sub):
    idx_fm = indices.astype(jnp.int32).T.reshape(-1)
    t_cm = emb_table.T
    x = _make_gather()(idx_fm, t_cm)
    wdT = jnp.transpose(w_deep.reshape(_F, _D, 16), (0, 2, 1))
    logits_cT, logits_sT = _dense(
        x,
        wdT,
        b_deep.reshape(16, 1),
        w_course[:16].T,
        w_course[16].reshape(728, 1),
        b_course.reshape(728, 1),
        w_sub[:16].T,
        w_sub[16].reshape(92, 1),
        b_sub.reshape(92, 1),
    )
    return (logits_cT.T, logits_sT.T)


# speedup vs baseline: 3.0500x; 1.4489x over previous
import functools

import jax
import jax.numpy as jnp
from jax import lax
from jax.experimental import pallas as pl
from jax.experimental.pallas import tpu as pltpu
from jax.experimental.pallas import tpu_sc as plsc

_B, _F, _V, _D = 4096, 26, 100000, 64
_N = _B * _F
_NC, _NS = 2, 16
_NW = _NC * _NS
_DPW = _D // _NW
_GRP = _B // 16
_UNROLL = 16
_NIB = 2
_NVB = 3


def _make_gather():
    mesh = plsc.VectorSubcoreMesh(core_axis_name="c", subcore_axis_name="s")

    @functools.partial(
        pl.kernel,
        mesh=mesh,
        compiler_params=pltpu.CompilerParams(needs_layout_passes=False),
        out_type=jax.ShapeDtypeStruct((_F, _D, _B), jnp.float32),
        scratch_types=(
            [pltpu.VMEM((_V,), jnp.float32)]
            + [pltpu.MemorySpace.VMEM_SHARED((_N,), jnp.int32)]
            + [pltpu.VMEM((_B,), jnp.int32)] * _NIB
            + [pltpu.VMEM((_B,), jnp.float32)] * _NVB
            + [pltpu.SemaphoreType.DMA] * (_NIB + _NVB)
        ),
    )
    def gather(idx_hbm, table_hbm, out_hbm, row_v, idx_sh, *bufs):
        idx_v = bufs[:_NIB]
        val_v = bufs[_NIB:_NIB + _NVB]
        isem = bufs[_NIB + _NVB:2 * _NIB + _NVB]
        osem = bufs[2 * _NIB + _NVB:]
        wid = lax.axis_index("s") * _NC + lax.axis_index("c")
        @pl.when(lax.axis_index("s") == 0)
        def _stage_idx():
            pltpu.sync_copy(idx_hbm, idx_sh)
        plsc.subcore_barrier()
        for r in range(_DPW):
            d = wid * _DPW + r
            pltpu.sync_copy(table_hbm.at[d], row_v)
            ih = [None] * _NIB
            oh = [None] * _NVB
            for p in range(_NIB - 1):
                ih[p] = pltpu.async_copy(
                    idx_sh.at[pl.ds(p * _B, _B)], idx_v[p], isem[p])
            for f in range(_F):
                bi = f % _NIB
                vi = f % _NVB
                nf = f + _NIB - 1
                if nf < _F:
                    ni = nf % _NIB
                    ih[ni] = pltpu.async_copy(
                        idx_sh.at[pl.ds(nf * _B, _B)],
                        idx_v[ni], isem[ni])
                ih[bi].wait()
                if oh[vi] is not None:
                    oh[vi].wait()

                @plsc.parallel_loop(0, _GRP, 1, unroll=_UNROLL)
                def _gather_body(g):
                    off = g * 16
                    iv = idx_v[bi][pl.ds(off, 16)]
                    val_v[vi][pl.ds(off, 16)] = plsc.load_gather(
                        row_v, [iv])
                oh[vi] = pltpu.async_copy(
                    val_v[vi], out_hbm.at[f].at[d], osem[vi])
            for vi in range(_NVB):
                if oh[vi] is not None:
                    oh[vi].wait()

    return gather


_make_gather = functools.cache(_make_gather)

_BB = 2048


def _dense_body(x_ref, wd_ref, bd_ref, wcm_ref, wcf_ref, bc_ref,
                wsm_ref, wsf_ref, bs_ref, outc_ref, outs_ref):
    deep = jnp.zeros((16, _BB), dtype=jnp.float32)
    t1 = jnp.zeros((_D, _BB), dtype=jnp.float32)
    t2 = jnp.zeros((_D, _BB), dtype=jnp.float32)
    for f in range(_F):
        xf = x_ref[f]
        deep = deep + jnp.dot(wd_ref[f], xf,
                              preferred_element_type=jnp.float32)
        t1 = t1 + xf
        t2 = t2 + xf * xf
    deep = jnp.maximum(deep + bd_ref[...], 0.0)
    s1 = jnp.sum(t1, axis=0, keepdims=True)
    s2 = jnp.sum(t2, axis=0, keepdims=True)
    cross = 0.5 * (s1 * s1 - s2)
    zc = (jnp.dot(wcm_ref[...], deep, preferred_element_type=jnp.float32)
          + wcf_ref[...] * cross + bc_ref[...])
    outc_ref[...] = 1.0 / (1.0 + jnp.exp(-zc))
    zs = (jnp.dot(wsm_ref[...], deep, preferred_element_type=jnp.float32)
          + wsf_ref[...] * cross + bs_ref[...])
    outs_ref[...] = 1.0 / (1.0 + jnp.exp(-zs))


def _dense(x, wdT, bdT, wcmT, wcfT, bcT, wsmT, wsfT, bsT):
    grid = (_B // _BB,)
    full2 = lambda shape: pl.BlockSpec(shape, lambda i: (0, 0))
    return pl.pallas_call(
        _dense_body,
        grid=grid,
        in_specs=[
            pl.BlockSpec((_F, _D, _BB), lambda i: (0, 0, i)),
            pl.BlockSpec(wdT.shape, lambda i: (0, 0, 0)),
            full2(bdT.shape),
            full2(wcmT.shape),
            full2(wcfT.shape),
            full2(bcT.shape),
            full2(wsmT.shape),
            full2(wsfT.shape),
            full2(bsT.shape),
        ],
        out_specs=[
            pl.BlockSpec((728, _BB), lambda i: (0, i)),
            pl.BlockSpec((92, _BB), lambda i: (0, i)),
        ],
        out_shape=[
            jax.ShapeDtypeStruct((728, _B), jnp.float32),
            jax.ShapeDtypeStruct((92, _B), jnp.float32),
        ],
    )(x, wdT, bdT, wcmT, wcfT, bcT, wsmT, wsfT, bsT)


def kernel(indices, emb_table, w_deep, b_deep, w_course, b_course, w_sub, b_sub):
    idx_fm = indices.astype(jnp.int32).T.reshape(-1)
    t_cm = emb_table.T
    x = _make_gather()(idx_fm, t_cm)
    wdT = jnp.transpose(w_deep.reshape(_F, _D, 16), (0, 2, 1))
    logits_cT, logits_sT = _dense(
        x,
        wdT,
        b_deep.reshape(16, 1),
        w_course[:16].T,
        w_course[16].reshape(728, 1),
        b_course.reshape(728, 1),
        w_sub[:16].T,
        w_sub[16].reshape(92, 1),
        b_sub.reshape(92, 1),
    )
    return (logits_cT.T, logits_sT.T)
